# baseline (speedup 1.0000x reference)
import jax
import jax.numpy as jnp
import numpy as np
from jax import lax
from jax.experimental import pallas as pl
from jax.experimental.pallas import tpu as pltpu
from jax.experimental.pallas import tpu_sc as plsc

M = 16384
S = 5
NUM_SUBCORES = 16
CHUNK = M // NUM_SUBCORES
VPC = CHUNK // 16
BITS_PER_TILE = 2 * S * CHUNK
NSAMP = S * VPC
MARGIN = 1.0

_mesh = plsc.VectorSubcoreMesh(core_axis_name="c", subcore_axis_name="s",
                               num_cores=1)


def _tf_hash(k1, k2, c1, c2):
    k1 = np.uint32(k1)
    k2 = np.uint32(k2)
    ks = [k1, k2, np.uint32(k1 ^ k2 ^ np.uint32(0x1BD11BDA))]
    rot = [(13, 15, 26, 6), (17, 29, 16, 24)]
    x0 = (np.asarray(c1, np.uint32) + k1).astype(np.uint32)
    x1 = (np.asarray(c2, np.uint32) + k2).astype(np.uint32)
    for g in range(5):
        for r in rot[g % 2]:
            x0 = (x0 + x1).astype(np.uint32)
            x1 = ((x1 << np.uint32(r)) | (x1 >> np.uint32(32 - r))).astype(np.uint32)
            x1 = (x1 ^ x0).astype(np.uint32)
        x0 = (x0 + ks[(g + 1) % 3]).astype(np.uint32)
        x1 = (x1 + ks[(g + 2) % 3] + np.uint32(g + 1)).astype(np.uint32)
    return x0, x1


def _randint_bits_np():
    base = np.array([0, 42], dtype=np.uint32)
    s0, s1 = _tf_hash(base[0], base[1],
                      np.zeros(2, np.uint32), np.arange(2, dtype=np.uint32))
    n = M * S
    zeros = np.zeros(n, np.uint32)
    iota = np.arange(n, dtype=np.uint32)
    hx0, hx1 = _tf_hash(s0[0], s1[0], zeros, iota)
    lx0, lx1 = _tf_hash(s0[1], s1[1], zeros, iota)
    hb = (hx0 ^ hx1).reshape(M, S).T.reshape(S, NUM_SUBCORES, CHUNK)
    lb = (lx0 ^ lx1).reshape(M, S).T.reshape(S, NUM_SUBCORES, CHUNK)
    bits = np.stack([hb, lb], 0)
    return np.ascontiguousarray(bits.transpose(2, 1, 0, 3)).reshape(-1)


_BITS = _randint_bits_np()


def _body(scores_hbm, target_hbm, bits_hbm, loss_hbm,
          tgt_v, sc_v, idx_v, cshared, cnt_sh, red_sh, negv, posv, bitv,
          sampv, pmv, redv, cntall_v, pv, accv, cw_v, sem0, sem1, sem2):
    sid = lax.axis_index("s")
    base = sid * CHUNK
    iot = lax.iota(jnp.int32, 16)

    bits_d = pltpu.async_copy(
        bits_hbm.at[pl.ds(sid * BITS_PER_TILE, BITS_PER_TILE)], bitv, sem1)
    tgt_d = pltpu.async_copy(target_hbm.at[pl.ds(base, CHUNK)], tgt_v, sem0)
    sc_d = [pltpu.async_copy(scores_hbm.at[pl.ds(base + q * 128, 128)],
                             sc_v.at[q], sem2)
            for q in range(8)]
    tgt_d.wait()

    zero = jnp.zeros((16,), jnp.int32)
    cnt = zero
    for k in range(VPC):
        cnt = cnt + tgt_v[pl.ds(k * 16, 16)]
    my_count = jnp.sum(cnt)
    cw_v[...] = jnp.where(iot == sid, my_count, 0)
    pltpu.sync_copy(cw_v, cnt_sh.at[pl.ds(sid * 16, 16)])

    plsc.subcore_barrier()

    pltpu.sync_copy(cnt_sh, cntall_v)
    counts = zero
    for j in range(NUM_SUBCORES):
        counts = counts + cntall_v[pl.ds(j * 16, 16)]
    pos_before = jnp.sum(counts * (iot < sid).astype(jnp.int32))
    p_total = jnp.sum(counts)

    P = jnp.broadcast_to(p_total, (16,))
    s_i = jnp.maximum(M - P, 1)
    s_u = plsc.bitcast(s_i, jnp.uint32)
    s_f = s_i.astype(jnp.float32)
    rcp = 1.0 / s_f
    m1 = lax.rem(jnp.full((16,), 65536, jnp.uint32), s_u)
    m2 = lax.rem(m1 * m1, s_u)
    a3 = lax.rem(m2 * m1, s_u)
    off = s_i * (2 + lax.div(1024 + s_i - 1, s_i))
    lim = jnp.full((16,), 2.0e9, jnp.float32)
    big = jnp.full((16,), 4294967296.0, jnp.float32)
    mask16 = jnp.full((16,), 0xFFFF, jnp.uint32)
    is_one = s_i == 1

    carry = jnp.int32(0)
    for k in range(VPC):
        t = tgt_v[pl.ds(k * 16, 16)]
        csum = plsc.cumsum(t)
        excl = csum - t
        prank = pos_before + carry + excl
        gidx = base + k * 16 + iot
        dest = jnp.where(t == 1, prank, M + gidx - prank)
        idx_v[k // 8, pl.ds((k % 8) * 16, 16)] = dest
        carry = carry + csum[15]

    for d in sc_d:
        d.wait()
    scat_d = [pltpu.async_copy(sc_v.at[q], cshared.at[idx_v.at[q]], sem2)
              for q in range(8)]
    for d in scat_d:
        d.wait()

    plsc.subcore_barrier()

    neg_copy = pltpu.async_copy(cshared.at[pl.ds(M, M)], negv, sem0)
    pos_copy = pltpu.async_copy(cshared.at[pl.ds(base, CHUNK)], posv, sem2)

    bits_d.wait()

    def samp_body(i, _):
        sc = i // VPC
        k = i - sc * VPC
        hb = bitv[pl.ds(sc * 2 * CHUNK + k * 16, 16)]
        lb = bitv[pl.ds(sc * 2 * CHUNK + CHUNK + k * 16, 16)]
        h1 = lax.shift_right_logical(hb, jnp.uint32(16))
        h0 = hb & mask16
        l1 = lax.shift_right_logical(lb, jnp.uint32(16))
        l0 = lb & mask16
        v = h1 * a3 + h0 * m2 + l1 * m1 + l0
        vi = plsc.bitcast(v, jnp.int32)
        vf = vi.astype(jnp.float32)
        vf = jnp.where(vi < 0, vf + big, vf)
        q1 = jnp.minimum(vf * rcp, lim).astype(jnp.int32)
        r1u = v - plsc.bitcast(q1, jnp.uint32) * s_u
        r1 = plsc.bitcast(r1u, jnp.int32) + off
        q2 = (r1.astype(jnp.float32) * rcp).astype(jnp.int32)
        r2 = r1 - q2 * s_i
        r2 = jnp.where(r2 < 0, r2 + s_i, r2)
        r2 = jnp.where(r2 >= s_i, r2 - s_i, r2)
        sampv[pl.ds(i * 16, 16)] = jnp.where(is_one, 0, r2)
        return 0

    lax.fori_loop(0, NSAMP, samp_body, 0)

    pos_copy.wait()
    neg_inf = jnp.full((16,), -3.0e38, jnp.float32)
    for k in range(VPC):
        rowid = base + k * 16 + iot
        pmv[pl.ds(k * 16, 16)] = jnp.where(
            rowid < P, MARGIN - posv[pl.ds(k * 16, 16)], neg_inf)

    neg_copy.wait()

    def gather_body(i, acc):
        sc = i // VPC
        k = i - sc * VPC
        samp = sampv[pl.ds(i * 16, 16)]
        neg = plsc.load_gather(negv, [samp])
        pm = pmv[pl.ds(k * 16, 16)]
        return acc + jnp.maximum(pm + neg, 0.0)

    acc = lax.fori_loop(0, NSAMP, gather_body, jnp.zeros((16,), jnp.float32))
    accv[...] = acc
    pltpu.sync_copy(accv, red_sh.at[pl.ds(sid * 16, 16)])

    plsc.subcore_barrier()

    @pl.when(sid == 0)
    def _():
        pltpu.sync_copy(red_sh, redv)
        tot = jnp.zeros((16,), jnp.float32)
        for i in range(NUM_SUBCORES):
            tot = tot + redv[pl.ds(i * 16, 16)]
        total_v = jnp.broadcast_to(jnp.sum(tot), (16,))
        denom_v = (P * S).astype(jnp.float32)
        pv[...] = total_v / denom_v
        pltpu.sync_copy(pv, loss_hbm)


_fused = pl.kernel(
    _body,
    out_type=jax.ShapeDtypeStruct((16,), jnp.float32),
    mesh=_mesh,
    compiler_params=pltpu.CompilerParams(needs_layout_passes=False),
    scratch_types=[
        pltpu.VMEM((CHUNK,), jnp.int32),
        pltpu.VMEM((8, 128), jnp.float32),
        pltpu.VMEM((8, 128), jnp.int32),
        pltpu.VMEM_SHARED((2 * M,), jnp.float32),
        pltpu.VMEM_SHARED((NUM_SUBCORES * 16,), jnp.int32),
        pltpu.VMEM_SHARED((NUM_SUBCORES * 16,), jnp.float32),
        pltpu.VMEM((M,), jnp.float32),
        pltpu.VMEM((CHUNK,), jnp.float32),
        pltpu.VMEM((BITS_PER_TILE,), jnp.uint32),
        pltpu.VMEM((NSAMP * 16,), jnp.int32),
        pltpu.VMEM((CHUNK,), jnp.float32),
        pltpu.VMEM((NUM_SUBCORES * 16,), jnp.float32),
        pltpu.VMEM((NUM_SUBCORES * 16,), jnp.int32),
        pltpu.VMEM((16,), jnp.float32),
        pltpu.VMEM((16,), jnp.float32),
        pltpu.VMEM((16,), jnp.int32),
        pltpu.SemaphoreType.DMA,
        pltpu.SemaphoreType.DMA,
        pltpu.SemaphoreType.DMA,
    ],
)


def kernel(scores, target):
    bits = jnp.asarray(_BITS)
    loss = _fused(scores, target, bits)
    return loss[0]
